# baseline (speedup 1.0000x reference)
import functools

import jax
import jax.numpy as jnp
from jax import lax
from jax.experimental import pallas as pl
from jax.experimental.pallas import tpu as pltpu

N = 10000
E = 160000
HEADS = 4
D = 256
QKV = HEADS * D
CAT = 3 * QKV + D

BN = 512
NPAD = 10240


def _mm_body(x_ref, w_ref, b_ref, o_ref):
    o_ref[...] = jnp.dot(x_ref[...], w_ref[...],
                         preferred_element_type=jnp.float32) + b_ref[0:1, :]


def _fused_mm(h, wcat, bcat8):
    din = h.shape[1]
    return pl.pallas_call(
        _mm_body,
        grid=(NPAD // BN,),
        in_specs=[
            pl.BlockSpec((BN, din), lambda i: (i, 0)),
            pl.BlockSpec((din, CAT), lambda i: (0, 0)),
            pl.BlockSpec((8, CAT), lambda i: (0, 0)),
        ],
        out_specs=pl.BlockSpec((BN, CAT), lambda i: (i, 0)),
        out_shape=jax.ShapeDtypeStruct((NPAD, CAT), jnp.float32),
    )(h, wcat, bcat8)


def _epi_mm_body(msg_ref, skip_ref, w_ref, b_ref, o_ref):
    m = (msg_ref[:, 0:D] + msg_ref[:, D:2 * D] + msg_ref[:, 2 * D:3 * D]
         + msg_ref[:, 3 * D:4 * D]) * 0.25
    h = jnp.maximum(m + skip_ref[...], 0.0)
    o_ref[...] = jnp.dot(h, w_ref[...],
                         preferred_element_type=jnp.float32) + b_ref[0:1, :]


def _epi_fused_mm(msg, skip, wcat, bcat8):
    return pl.pallas_call(
        _epi_mm_body,
        grid=(NPAD // BN,),
        in_specs=[
            pl.BlockSpec((BN, QKV), lambda i: (i, 0)),
            pl.BlockSpec((BN, D), lambda i: (i, 0)),
            pl.BlockSpec((D, CAT), lambda i: (0, 0)),
            pl.BlockSpec((8, CAT), lambda i: (0, 0)),
        ],
        out_specs=pl.BlockSpec((BN, CAT), lambda i: (i, 0)),
        out_shape=jax.ShapeDtypeStruct((NPAD, CAT), jnp.float32),
    )(msg, skip, wcat, bcat8)


def _final_body(msg_ref, skip_ref, o_ref):
    m = (msg_ref[:, 0:D] + msg_ref[:, D:2 * D] + msg_ref[:, 2 * D:3 * D]
         + msg_ref[:, 3 * D:4 * D]) * 0.25
    h = jnp.maximum(m + skip_ref[...], 0.0)
    hmax = jnp.max(h, axis=1, keepdims=True)
    lse = jnp.log(jnp.sum(jnp.exp(h - hmax), axis=1, keepdims=True)) + hmax
    o_ref[...] = h - lse


def _final(msg, skip):
    return pl.pallas_call(
        _final_body,
        grid=(NPAD // BN,),
        in_specs=[
            pl.BlockSpec((BN, QKV), lambda i: (i, 0)),
            pl.BlockSpec((BN, D), lambda i: (i, 0)),
        ],
        out_specs=pl.BlockSpec((BN, D), lambda i: (i, 0)),
        out_shape=jax.ShapeDtypeStruct((NPAD, D), jnp.float32),
    )(msg, skip)


def _edge_phase(q, k, v, src, dst):
    alpha = (q[dst] * k[src]).sum(-1) * (1.0 / jnp.sqrt(jnp.float32(D)))
    amax = jax.ops.segment_max(alpha, dst, num_segments=N)
    amax = jnp.where(jnp.isfinite(amax), amax, 0.0)
    ex = jnp.exp(alpha - amax[dst])
    denom = jax.ops.segment_sum(ex, dst, num_segments=N)
    attn = ex / (denom[dst] + 1e-16)
    msg = v[src] * attn[:, :, None]
    out = jax.ops.segment_sum(msg, dst, num_segments=N)
    return out.reshape(N, QKV)


def kernel(x, edge_index,
           Wq0, bq0, Wk0, bk0, Wv0, bv0, Ws0, bs0,
           Wq1, bq1, Wk1, bk1, Wv1, bv1, Ws1, bs1,
           Wq2, bq2, Wk2, bk2, Wv2, bv2, Ws2, bs2):
    src = edge_index[0]
    dst = edge_index[1]

    def cat_w(Wq, Wk, Wv, Ws, bq, bk, bv, bs):
        w = jnp.concatenate([Wq, Wk, Wv, Ws], axis=1)
        b = jnp.concatenate([bq, bk, bv, bs], axis=0)
        b8 = jnp.zeros((8, CAT), jnp.float32).at[0].set(b)
        return w, b8

    w0, b0 = cat_w(Wq0, Wk0, Wv0, Ws0, bq0, bk0, bv0, bs0)
    w1, b1 = cat_w(Wq1, Wk1, Wv1, Ws1, bq1, bk1, bv1, bs1)
    w2, b2 = cat_w(Wq2, Wk2, Wv2, Ws2, bq2, bk2, bv2, bs2)

    xp = jnp.zeros((NPAD, D), jnp.float32).at[:N].set(x)

    def split(o):
        q = o[:N, 0:QKV].reshape(N, HEADS, D)
        k = o[:N, QKV:2 * QKV].reshape(N, HEADS, D)
        v = o[:N, 2 * QKV:3 * QKV].reshape(N, HEADS, D)
        s = o[:, 3 * QKV:CAT]
        return q, k, v, s

    o0 = _fused_mm(xp, w0, b0)
    q, k, v, s0 = split(o0)
    msg0 = _edge_phase(q, k, v, src, dst)
    msg0 = jnp.zeros((NPAD, QKV), jnp.float32).at[:N].set(msg0)

    o1 = _epi_fused_mm(msg0, s0, w1, b1)
    q, k, v, s1 = split(o1)
    msg1 = _edge_phase(q, k, v, src, dst)
    msg1 = jnp.zeros((NPAD, QKV), jnp.float32).at[:N].set(msg1)

    o2 = _epi_fused_mm(msg1, s1, w2, b2)
    q, k, v, s2 = split(o2)
    msg2 = _edge_phase(q, k, v, src, dst)
    msg2 = jnp.zeros((NPAD, QKV), jnp.float32).at[:N].set(msg2)

    return _final(msg2, s2)[:N]


# speedup vs baseline: 4.8870x; 4.8870x over previous
import functools

import jax
import jax.numpy as jnp
from jax import lax
from jax.experimental import pallas as pl
from jax.experimental.pallas import tpu as pltpu
from jax.experimental.pallas import tpu_sc as plsc

N = 10000
E = 160000
HEADS = 4
D = 256
QKV = HEADS * D
CAT = 3 * QKV + D

BN = 512
NPAD = 10240

NW = 32
NPW = NPAD // NW
CE = 32
EPAD = E + NW * CE


def _mm_body(x_ref, w_ref, b_ref, q_ref, k_ref, v_ref, s_ref):
    o = jnp.dot(x_ref[...], w_ref[...],
                preferred_element_type=jnp.float32) + b_ref[0:1, :]
    q_ref[...] = o[:, 0:QKV]
    k_ref[...] = o[:, QKV:2 * QKV]
    v_ref[...] = o[:, 2 * QKV:3 * QKV]
    s_ref[...] = o[:, 3 * QKV:CAT]


_MM_OUT = [
    jax.ShapeDtypeStruct((NPAD, QKV), jnp.float32),
    jax.ShapeDtypeStruct((NPAD, QKV), jnp.float32),
    jax.ShapeDtypeStruct((NPAD, QKV), jnp.float32),
    jax.ShapeDtypeStruct((NPAD, D), jnp.float32),
]

_MM_OUT_SPECS = [
    pl.BlockSpec((BN, QKV), lambda i: (i, 0)),
    pl.BlockSpec((BN, QKV), lambda i: (i, 0)),
    pl.BlockSpec((BN, QKV), lambda i: (i, 0)),
    pl.BlockSpec((BN, D), lambda i: (i, 0)),
]


def _fused_mm(h, wcat, bcat8):
    din = h.shape[1]
    return pl.pallas_call(
        _mm_body,
        grid=(NPAD // BN,),
        in_specs=[
            pl.BlockSpec((BN, din), lambda i: (i, 0)),
            pl.BlockSpec((din, CAT), lambda i: (0, 0)),
            pl.BlockSpec((8, CAT), lambda i: (0, 0)),
        ],
        out_specs=_MM_OUT_SPECS,
        out_shape=_MM_OUT,
    )(h, wcat, bcat8)


def _epi_mm_body(msg_ref, skip_ref, w_ref, b_ref, q_ref, k_ref, v_ref, s_ref):
    m = (msg_ref[:, 0:D] + msg_ref[:, D:2 * D] + msg_ref[:, 2 * D:3 * D]
         + msg_ref[:, 3 * D:4 * D]) * 0.25
    h = jnp.maximum(m + skip_ref[...], 0.0)
    o = jnp.dot(h, w_ref[...], preferred_element_type=jnp.float32) + b_ref[0:1, :]
    q_ref[...] = o[:, 0:QKV]
    k_ref[...] = o[:, QKV:2 * QKV]
    v_ref[...] = o[:, 2 * QKV:3 * QKV]
    s_ref[...] = o[:, 3 * QKV:CAT]


def _epi_fused_mm(msg, skip, wcat, bcat8):
    return pl.pallas_call(
        _epi_mm_body,
        grid=(NPAD // BN,),
        in_specs=[
            pl.BlockSpec((BN, QKV), lambda i: (i, 0)),
            pl.BlockSpec((BN, D), lambda i: (i, 0)),
            pl.BlockSpec((D, CAT), lambda i: (0, 0)),
            pl.BlockSpec((8, CAT), lambda i: (0, 0)),
        ],
        out_specs=_MM_OUT_SPECS,
        out_shape=_MM_OUT,
    )(msg, skip, wcat, bcat8)


def _final_body(msg_ref, skip_ref, o_ref):
    m = (msg_ref[:, 0:D] + msg_ref[:, D:2 * D] + msg_ref[:, 2 * D:3 * D]
         + msg_ref[:, 3 * D:4 * D]) * 0.25
    h = jnp.maximum(m + skip_ref[...], 0.0)
    hmax = jnp.max(h, axis=1, keepdims=True)
    lse = jnp.log(jnp.sum(jnp.exp(h - hmax), axis=1, keepdims=True)) + hmax
    o_ref[...] = h - lse


def _final(msg, skip):
    return pl.pallas_call(
        _final_body,
        grid=(NPAD // BN,),
        in_specs=[
            pl.BlockSpec((BN, QKV), lambda i: (i, 0)),
            pl.BlockSpec((BN, D), lambda i: (i, 0)),
        ],
        out_specs=pl.BlockSpec((BN, D), lambda i: (i, 0)),
        out_shape=jax.ShapeDtypeStruct((NPAD, D), jnp.float32),
    )(msg, skip)


_IOTA16 = functools.partial(lax.broadcasted_iota, jnp.int32, (16,), 0)


_GDN = lax.GatherDimensionNumbers(
    offset_dims=(), collapsed_slice_dims=(0,), start_index_map=(0,))


def _gather16(vec, idx16):
    return lax.gather(vec, idx16[:, None], _GDN, slice_sizes=(1,),
                      mode=lax.GatherScatterMode.PROMISE_IN_BOUNDS)


def _hsum16(a):
    io = _IOTA16()
    for sh in (8, 4, 2, 1):
        a = a + _gather16(a, io ^ sh)
    return a


def _edge_body(q_hbm, k_hbm, v_hbm, src_hbm, dst_hbm, rp_hbm,
               out_hbm, alpha_hbm,
               rpv, sv, dv, qi, rows_a, rows_b, abuf, marr, sarr,
               acc, zrow, sem_a, sem_b):
    nc = 2
    wid = lax.axis_index("s") * nc + lax.axis_index("c")
    n_lo = wid * NPW

    pltpu.sync_copy(rp_hbm.at[wid], rpv)
    rv = rpv[...]
    e_lo = rv[0]
    pcnt = rv[1]
    nchunks = pcnt // CE

    zero16 = jnp.zeros((16,), jnp.float32)

    def _init_ms(i, _):
        o = pl.multiple_of(i * 16, 8)
        marr[pl.ds(o, 16)] = jnp.full((16,), -3e38, jnp.float32)
        sarr[pl.ds(o, 16)] = zero16
        return 0
    lax.fori_loop(0, NPW, _init_ms, 0)

    def _init_row(i, _):
        acc[pl.ds(i * 16, 16)] = zero16
        zrow[pl.ds(i * 16, 16)] = zero16
        return 0
    lax.fori_loop(0, QKV // 16, _init_row, 0)

    def _chunk1(c, _):
        e0 = pl.multiple_of(e_lo + c * CE, CE)
        pltpu.sync_copy(src_hbm.at[pl.ds(e0, CE)], sv)
        pltpu.sync_copy(dst_hbm.at[pl.ds(e0, CE)], dv)
        qi[pl.ds(0, 16)] = jnp.maximum(dv[pl.ds(0, 16)], 0)
        qi[pl.ds(16, 16)] = jnp.maximum(dv[pl.ds(16, 16)], 0)
        pltpu.async_copy(k_hbm.at[sv], rows_a, sem_a).wait()
        pltpu.async_copy(q_hbm.at[qi], rows_b, sem_b).wait()

        def _edge1(e, _):
            de = dv[pl.ds(e, 1)][0]
            valid = de >= 0
            li = jnp.clip(de - n_lo, 0, NPW - 1)

            av = zero16
            io = _IOTA16()
            for h in range(HEADS):
                a = zero16
                for j in range(16):
                    off = h * D + j * 16
                    a = a + rows_b[e, pl.ds(off, 16)] * rows_a[e, pl.ds(off, 16)]
                a = a * 0.0625
                av = jnp.where(io == h, _hsum16(a), av)

            lo = pl.multiple_of(li * 16, 8)
            eo = pl.multiple_of(e * 16, 8)
            mrow = marr[pl.ds(lo, 16)]
            srow = sarr[pl.ds(lo, 16)]
            mnew = jnp.maximum(mrow, av)
            snew = srow * jnp.exp(mrow - mnew) + jnp.exp(av - mnew)

            @pl.when(valid)
            def _():
                marr[pl.ds(lo, 16)] = mnew
                sarr[pl.ds(lo, 16)] = snew
                abuf[pl.ds(eo, 16)] = av
            return 0

        lax.fori_loop(0, CE, _edge1, 0)
        pltpu.sync_copy(abuf, alpha_hbm.at[pl.ds(e0 * 16, CE * 16)])
        return 0

    lax.fori_loop(0, nchunks, _chunk1, 0)

    def _zero_acc():
        def _za(i, _):
            acc[pl.ds(i * 16, 16)] = zero16
            return 0
        lax.fori_loop(0, QKV // 16, _za, 0)

    def _chunk2(c, cur):
        e0 = pl.multiple_of(e_lo + c * CE, CE)
        pltpu.sync_copy(src_hbm.at[pl.ds(e0, CE)], sv)
        pltpu.sync_copy(dst_hbm.at[pl.ds(e0, CE)], dv)
        pltpu.sync_copy(alpha_hbm.at[pl.ds(e0 * 16, CE * 16)], abuf)
        pltpu.async_copy(v_hbm.at[sv], rows_a, sem_a).wait()

        def _edge2(e, cur):
            de = dv[pl.ds(e, 1)][0]
            valid = de >= 0
            li = jnp.clip(de - n_lo, 0, NPW - 1)
            changed = jnp.logical_and(valid, de != cur)

            @pl.when(changed)
            def _():
                @pl.when(cur >= n_lo)
                def _():
                    pltpu.sync_copy(acc, out_hbm.at[cur])
                    _zero_acc()

                def _zfill(m, _):
                    pltpu.sync_copy(zrow, out_hbm.at[m])
                    return 0
                lax.fori_loop(cur + 1, de, _zfill, 0)

            cur2 = jnp.where(changed, de, cur)

            @pl.when(valid)
            def _():
                lo = pl.multiple_of(li * 16, 8)
                eo = pl.multiple_of(e * 16, 8)
                arow = abuf[pl.ds(eo, 16)]
                mrow = marr[pl.ds(lo, 16)]
                srow = sarr[pl.ds(lo, 16)]
                attn = jnp.exp(arow - mrow) / (srow + 1e-16)
                for h in range(HEADS):
                    bc = _gather16(attn, jnp.full((16,), h, jnp.int32))
                    for j in range(16):
                        off = h * D + j * 16
                        acc[pl.ds(off, 16)] += bc * rows_a[e, pl.ds(off, 16)]
            return cur2

        return lax.fori_loop(0, CE, _edge2, cur)

    cur = lax.fori_loop(0, nchunks, _chunk2, n_lo - 1)

    @pl.when(cur >= n_lo)
    def _():
        pltpu.sync_copy(acc, out_hbm.at[cur])

    def _zfill_tail(m, _):
        pltpu.sync_copy(zrow, out_hbm.at[m])
        return 0
    lax.fori_loop(cur + 1, n_lo + NPW, _zfill_tail, 0)


def _edge_sc(q, k, v, src_flat, dst_flat, rp):
    mesh = plsc.VectorSubcoreMesh(core_axis_name="c", subcore_axis_name="s")
    f = pl.kernel(
        _edge_body,
        mesh=mesh,
        out_type=[
            jax.ShapeDtypeStruct((NPAD, QKV), jnp.float32),
            jax.ShapeDtypeStruct((EPAD * 16,), jnp.float32),
        ],
        scratch_types=[
            pltpu.VMEM((16,), jnp.int32),
            pltpu.VMEM((CE,), jnp.int32),
            pltpu.VMEM((CE,), jnp.int32),
            pltpu.VMEM((CE,), jnp.int32),
            pltpu.VMEM((CE, QKV), jnp.float32),
            pltpu.VMEM((CE, QKV), jnp.float32),
            pltpu.VMEM((CE * 16,), jnp.float32),
            pltpu.VMEM((NPW * 16,), jnp.float32),
            pltpu.VMEM((NPW * 16,), jnp.float32),
            pltpu.VMEM((QKV,), jnp.float32),
            pltpu.VMEM((QKV,), jnp.float32),
            pltpu.SemaphoreType.DMA,
            pltpu.SemaphoreType.DMA,
        ],
    )
    msg, _alpha = f(q, k, v, src_flat, dst_flat, rp)
    return msg


def _prep_edges(src, dst):
    order = jnp.argsort(dst)
    dst_s = jnp.take(dst, order)
    src_s = jnp.take(src, order)
    bounds = jnp.searchsorted(dst_s, jnp.arange(NW + 1, dtype=jnp.int32) * NPW)
    bounds = bounds.astype(jnp.int32)
    cnt = bounds[1:] - bounds[:-1]
    pcnt = ((cnt + CE - 1) // CE) * CE
    starts = jnp.concatenate(
        [jnp.zeros((1,), jnp.int32), jnp.cumsum(pcnt).astype(jnp.int32)])

    p = jnp.arange(EPAD, dtype=jnp.int32)
    wb = jnp.clip(jnp.searchsorted(starts, p, side="right").astype(jnp.int32) - 1,
                  0, NW - 1)
    loc = p - starts[wb]
    valid = loc < cnt[wb]
    ep = jnp.clip(bounds[wb] + loc, 0, E - 1)
    src_flat = jnp.where(valid, jnp.take(src_s, ep), 0)
    dst_flat = jnp.where(valid, jnp.take(dst_s, ep), -1)

    rp = jnp.zeros((NW, 16), jnp.int32)
    rp = rp.at[:, 0].set(starts[:NW])
    rp = rp.at[:, 1].set(pcnt)
    return src_flat, dst_flat, rp


def kernel(x, edge_index,
           Wq0, bq0, Wk0, bk0, Wv0, bv0, Ws0, bs0,
           Wq1, bq1, Wk1, bk1, Wv1, bv1, Ws1, bs1,
           Wq2, bq2, Wk2, bk2, Wv2, bv2, Ws2, bs2):
    src = edge_index[0]
    dst = edge_index[1]
    src_flat, dst_flat, rp = _prep_edges(src, dst)

    def cat_w(Wq, Wk, Wv, Ws, bq, bk, bv, bs):
        w = jnp.concatenate([Wq, Wk, Wv, Ws], axis=1)
        b = jnp.concatenate([bq, bk, bv, bs], axis=0)
        b8 = jnp.zeros((8, CAT), jnp.float32).at[0].set(b)
        return w, b8

    w0, b0 = cat_w(Wq0, Wk0, Wv0, Ws0, bq0, bk0, bv0, bs0)
    w1, b1 = cat_w(Wq1, Wk1, Wv1, Ws1, bq1, bk1, bv1, bs1)
    w2, b2 = cat_w(Wq2, Wk2, Wv2, Ws2, bq2, bk2, bv2, bs2)

    xp = jnp.zeros((NPAD, D), jnp.float32).at[:N].set(x)

    q, k, v, s0 = _fused_mm(xp, w0, b0)
    msg0 = _edge_sc(q, k, v, src_flat, dst_flat, rp)

    q, k, v, s1 = _epi_fused_mm(msg0, s0, w1, b1)
    msg1 = _edge_sc(q, k, v, src_flat, dst_flat, rp)

    q, k, v, s2 = _epi_fused_mm(msg1, s1, w2, b2)
    msg2 = _edge_sc(q, k, v, src_flat, dst_flat, rp)

    return _final(msg2, s2)[:N]


# speedup vs baseline: 6.1422x; 1.2569x over previous
import functools

import jax
import jax.numpy as jnp
from jax import lax
from jax.experimental import pallas as pl
from jax.experimental.pallas import tpu as pltpu
from jax.experimental.pallas import tpu_sc as plsc

N = 10000
E = 160000
HEADS = 4
D = 256
QKV = HEADS * D
CAT = 3 * QKV + D

BN = 512
NPAD = 10240

NW = 32
NPW = NPAD // NW
CE = 16
WIN = 16
NWIN = NPW // WIN
EPAD = E + NW * CE


def _mm_body(x_ref, w_ref, b_ref, q_ref, k_ref, v_ref, s_ref):
    o = jnp.dot(x_ref[...], w_ref[...],
                preferred_element_type=jnp.float32) + b_ref[0:1, :]
    q_ref[...] = o[:, 0:QKV]
    k_ref[...] = o[:, QKV:2 * QKV]
    v_ref[...] = o[:, 2 * QKV:3 * QKV]
    s_ref[...] = o[:, 3 * QKV:CAT]


_MM_OUT = [
    jax.ShapeDtypeStruct((NPAD, QKV), jnp.float32),
    jax.ShapeDtypeStruct((NPAD, QKV), jnp.float32),
    jax.ShapeDtypeStruct((NPAD, QKV), jnp.float32),
    jax.ShapeDtypeStruct((NPAD, D), jnp.float32),
]

_MM_OUT_SPECS = [
    pl.BlockSpec((BN, QKV), lambda i: (i, 0)),
    pl.BlockSpec((BN, QKV), lambda i: (i, 0)),
    pl.BlockSpec((BN, QKV), lambda i: (i, 0)),
    pl.BlockSpec((BN, D), lambda i: (i, 0)),
]


def _fused_mm(h, wcat, bcat8):
    din = h.shape[1]
    return pl.pallas_call(
        _mm_body,
        grid=(NPAD // BN,),
        in_specs=[
            pl.BlockSpec((BN, din), lambda i: (i, 0)),
            pl.BlockSpec((din, CAT), lambda i: (0, 0)),
            pl.BlockSpec((8, CAT), lambda i: (0, 0)),
        ],
        out_specs=_MM_OUT_SPECS,
        out_shape=_MM_OUT,
    )(h, wcat, bcat8)


def _epi_mm_body(msg_ref, skip_ref, w_ref, b_ref, q_ref, k_ref, v_ref, s_ref):
    m = (msg_ref[:, 0:D] + msg_ref[:, D:2 * D] + msg_ref[:, 2 * D:3 * D]
         + msg_ref[:, 3 * D:4 * D]) * 0.25
    h = jnp.maximum(m + skip_ref[...], 0.0)
    o = jnp.dot(h, w_ref[...], preferred_element_type=jnp.float32) + b_ref[0:1, :]
    q_ref[...] = o[:, 0:QKV]
    k_ref[...] = o[:, QKV:2 * QKV]
    v_ref[...] = o[:, 2 * QKV:3 * QKV]
    s_ref[...] = o[:, 3 * QKV:CAT]


def _epi_fused_mm(msg, skip, wcat, bcat8):
    return pl.pallas_call(
        _epi_mm_body,
        grid=(NPAD // BN,),
        in_specs=[
            pl.BlockSpec((BN, QKV), lambda i: (i, 0)),
            pl.BlockSpec((BN, D), lambda i: (i, 0)),
            pl.BlockSpec((D, CAT), lambda i: (0, 0)),
            pl.BlockSpec((8, CAT), lambda i: (0, 0)),
        ],
        out_specs=_MM_OUT_SPECS,
        out_shape=_MM_OUT,
    )(msg, skip, wcat, bcat8)


def _final_body(msg_ref, skip_ref, o_ref):
    m = (msg_ref[:, 0:D] + msg_ref[:, D:2 * D] + msg_ref[:, 2 * D:3 * D]
         + msg_ref[:, 3 * D:4 * D]) * 0.25
    h = jnp.maximum(m + skip_ref[...], 0.0)
    hmax = jnp.max(h, axis=1, keepdims=True)
    lse = jnp.log(jnp.sum(jnp.exp(h - hmax), axis=1, keepdims=True)) + hmax
    o_ref[...] = h - lse


def _final(msg, skip):
    return pl.pallas_call(
        _final_body,
        grid=(NPAD // BN,),
        in_specs=[
            pl.BlockSpec((BN, QKV), lambda i: (i, 0)),
            pl.BlockSpec((BN, D), lambda i: (i, 0)),
        ],
        out_specs=pl.BlockSpec((BN, D), lambda i: (i, 0)),
        out_shape=jax.ShapeDtypeStruct((NPAD, D), jnp.float32),
    )(msg, skip)


_IOTA16 = functools.partial(lax.broadcasted_iota, jnp.int32, (16,), 0)


_GDN = lax.GatherDimensionNumbers(
    offset_dims=(), collapsed_slice_dims=(0,), start_index_map=(0,))


def _gather16(vec, idx16):
    return lax.gather(vec, idx16[:, None], _GDN, slice_sizes=(1,),
                      mode=lax.GatherScatterMode.PROMISE_IN_BOUNDS)


def _hsum16(a):
    io = _IOTA16()
    for sh in (8, 4, 2, 1):
        a = a + _gather16(a, io ^ sh)
    return a


def _edge_body(q_hbm, k_hbm, v_hbm, src_hbm, dst_hbm, rp_hbm,
               out_hbm, alpha_hbm,
               rpv, sv0, sv1, dv0, dv1, qi0, qi1,
               ka0, ka1, qb0, qb1, ab0, ab1, marr, sarr, win, zwin,
               sk0, sk1, sq0, sq1, sa0, sa1):
    nc = 2
    wid = lax.axis_index("s") * nc + lax.axis_index("c")
    n_lo = wid * NPW

    pltpu.sync_copy(rp_hbm.at[wid], rpv)
    rv = rpv[...]
    e_lo = rv[0]
    pcnt = rv[1]
    nchunks = pcnt // CE

    zero16 = jnp.zeros((16,), jnp.float32)

    def _init_ms(i, _):
        o = pl.multiple_of(i * 16, 8)
        marr[pl.ds(o, 16)] = jnp.full((16,), -3e38, jnp.float32)
        sarr[pl.ds(o, 16)] = zero16
        return 0
    lax.fori_loop(0, NPW, _init_ms, 0)

    def _init_win(i, _):
        for jj in range(QKV // 16):
            win[i, pl.ds(jj * 16, 16)] = zero16
            zwin[i, pl.ds(jj * 16, 16)] = zero16
        return 0
    lax.fori_loop(0, WIN, _init_win, 0)

    def _e0(c):
        return pl.multiple_of(e_lo + c * CE, CE)

    bufs = ((sv0, dv0, qi0, ka0, qb0, ab0, sk0, sq0, sa0),
            (sv1, dv1, qi1, ka1, qb1, ab1, sk1, sq1, sa1))

    def _fire1(c, b):
        sv, dv, qi, ka, qb = b[0], b[1], b[2], b[3], b[4]
        e0 = _e0(c)
        pltpu.sync_copy(src_hbm.at[pl.ds(e0, CE)], sv)
        pltpu.sync_copy(dst_hbm.at[pl.ds(e0, CE)], dv)
        qi[...] = jnp.maximum(dv[...], 0)
        pltpu.make_async_copy(k_hbm.at[sv], ka, b[6]).start()
        pltpu.make_async_copy(q_hbm.at[qi], qb, b[7]).start()

    def _wait1(b):
        pltpu.make_async_copy(k_hbm.at[b[0]], b[3], b[6]).wait()
        pltpu.make_async_copy(q_hbm.at[b[2]], b[4], b[7]).wait()

    def _compute1(c, b):
        dv, ka, qb, ab = b[1], b[3], b[4], b[5]
        e0 = _e0(c)

        def _edge1(e, _):
            de = dv[pl.ds(e, 1)][0]
            valid = de >= 0
            li = jnp.clip(de - n_lo, 0, NPW - 1)

            av = zero16
            io = _IOTA16()
            for h in range(HEADS):
                a = zero16
                for j in range(16):
                    off = h * D + j * 16
                    a = a + qb[e, pl.ds(off, 16)] * ka[e, pl.ds(off, 16)]
                a = a * 0.0625
                av = jnp.where(io == h, _hsum16(a), av)

            lo = pl.multiple_of(li * 16, 8)
            eo = pl.multiple_of(e * 16, 8)
            mrow = marr[pl.ds(lo, 16)]
            srow = sarr[pl.ds(lo, 16)]
            mnew = jnp.maximum(mrow, av)
            snew = srow * jnp.exp(mrow - mnew) + jnp.exp(av - mnew)

            @pl.when(valid)
            def _():
                marr[pl.ds(lo, 16)] = mnew
                sarr[pl.ds(lo, 16)] = snew
                ab[pl.ds(eo, 16)] = av
            return 0

        lax.fori_loop(0, CE, _edge1, 0)
        pltpu.sync_copy(ab, alpha_hbm.at[pl.ds(e0 * 16, CE * 16)])

    @pl.when(nchunks > 0)
    def _():
        _fire1(0, bufs[0])

    def _pair1(c2, _):
        c = c2 * 2

        @pl.when(c + 1 < nchunks)
        def _():
            _fire1(c + 1, bufs[1])
        _wait1(bufs[0])
        _compute1(c, bufs[0])

        @pl.when(c + 1 < nchunks)
        def _():
            @pl.when(c + 2 < nchunks)
            def _():
                _fire1(c + 2, bufs[0])
            _wait1(bufs[1])
            _compute1(c + 1, bufs[1])
        return 0

    lax.fori_loop(0, (nchunks + 1) // 2, _pair1, 0)

    def _zero_win():
        def _zw(i, _):
            for jj in range(QKV // 16):
                win[i, pl.ds(jj * 16, 16)] = zero16
            return 0
        lax.fori_loop(0, WIN, _zw, 0)

    def _fire2(c, b):
        sv, dv, ka, ab = b[0], b[1], b[3], b[5]
        e0 = _e0(c)
        pltpu.sync_copy(src_hbm.at[pl.ds(e0, CE)], sv)
        pltpu.sync_copy(dst_hbm.at[pl.ds(e0, CE)], dv)
        pltpu.make_async_copy(v_hbm.at[sv], ka, b[6]).start()
        pltpu.make_async_copy(alpha_hbm.at[pl.ds(e0 * 16, CE * 16)], ab,
                              b[8]).start()

    def _wait2(c, b):
        e0 = _e0(c)
        pltpu.make_async_copy(v_hbm.at[b[0]], b[3], b[6]).wait()
        pltpu.make_async_copy(alpha_hbm.at[pl.ds(e0 * 16, CE * 16)], b[5],
                              b[8]).wait()

    def _flush_win(wi):
        r0 = pl.multiple_of(n_lo + wi * WIN, WIN)
        pltpu.sync_copy(win, out_hbm.at[pl.ds(r0, WIN)])

    def _zero_fill(wlo, whi):
        def _zf(u, _):
            r0 = pl.multiple_of(n_lo + u * WIN, WIN)
            pltpu.sync_copy(zwin, out_hbm.at[pl.ds(r0, WIN)])
            return 0
        lax.fori_loop(wlo, whi, _zf, 0)

    def _compute2(c, b, wcur):
        dv, ka, ab = b[1], b[3], b[5]

        def _edge2(e, wcur):
            de = dv[pl.ds(e, 1)][0]
            valid = de >= 0
            li = jnp.clip(de - n_lo, 0, NPW - 1)
            wi = li // WIN
            changed = jnp.logical_and(valid, wi != wcur)

            @pl.when(changed)
            def _():
                @pl.when(wcur >= 0)
                def _():
                    _flush_win(wcur)
                    _zero_win()
                _zero_fill(jnp.maximum(wcur + 1, 0), wi)

            wcur2 = jnp.where(changed, wi, wcur)

            @pl.when(valid)
            def _():
                r = li - wi * WIN
                lo = pl.multiple_of(li * 16, 8)
                eo = pl.multiple_of(e * 16, 8)
                arow = ab[pl.ds(eo, 16)]
                mrow = marr[pl.ds(lo, 16)]
                srow = sarr[pl.ds(lo, 16)]
                attn = jnp.exp(arow - mrow) / (srow + 1e-16)
                for h in range(HEADS):
                    bc = _gather16(attn, jnp.full((16,), h, jnp.int32))
                    for j in range(16):
                        off = h * D + j * 16
                        win[r, pl.ds(off, 16)] += bc * ka[e, pl.ds(off, 16)]
            return wcur2

        return lax.fori_loop(0, CE, _edge2, wcur)

    @pl.when(nchunks > 0)
    def _():
        _fire2(0, bufs[0])

    def _pair2(c2, wcur):
        c = c2 * 2

        @pl.when(c + 1 < nchunks)
        def _():
            _fire2(c + 1, bufs[1])
        _wait2(c, bufs[0])
        wcur = _compute2(c, bufs[0], wcur)

        def _odd(wcur):
            @pl.when(c + 2 < nchunks)
            def _():
                _fire2(c + 2, bufs[0])
            _wait2(c + 1, bufs[1])
            return _compute2(c + 1, bufs[1], wcur)

        return lax.cond(c + 1 < nchunks, _odd, lambda w: w, wcur)

    wcur = lax.fori_loop(0, (nchunks + 1) // 2, _pair2, -1)

    @pl.when(wcur >= 0)
    def _():
        _flush_win(wcur)
    _zero_fill(jnp.maximum(wcur + 1, 0), NWIN)


def _edge_sc(q, k, v, src_flat, dst_flat, rp):
    mesh = plsc.VectorSubcoreMesh(core_axis_name="c", subcore_axis_name="s")
    f = pl.kernel(
        _edge_body,
        mesh=mesh,
        out_type=[
            jax.ShapeDtypeStruct((NPAD, QKV), jnp.float32),
            jax.ShapeDtypeStruct((EPAD * 16,), jnp.float32),
        ],
        scratch_types=[
            pltpu.VMEM((16,), jnp.int32),
            pltpu.VMEM((CE,), jnp.int32),
            pltpu.VMEM((CE,), jnp.int32),
            pltpu.VMEM((CE,), jnp.int32),
            pltpu.VMEM((CE,), jnp.int32),
            pltpu.VMEM((CE,), jnp.int32),
            pltpu.VMEM((CE,), jnp.int32),
            pltpu.VMEM((CE, QKV), jnp.float32),
            pltpu.VMEM((CE, QKV), jnp.float32),
            pltpu.VMEM((CE, QKV), jnp.float32),
            pltpu.VMEM((CE, QKV), jnp.float32),
            pltpu.VMEM((CE * 16,), jnp.float32),
            pltpu.VMEM((CE * 16,), jnp.float32),
            pltpu.VMEM((NPW * 16,), jnp.float32),
            pltpu.VMEM((NPW * 16,), jnp.float32),
            pltpu.VMEM((WIN, QKV), jnp.float32),
            pltpu.VMEM((WIN, QKV), jnp.float32),
            pltpu.SemaphoreType.DMA,
            pltpu.SemaphoreType.DMA,
            pltpu.SemaphoreType.DMA,
            pltpu.SemaphoreType.DMA,
            pltpu.SemaphoreType.DMA,
            pltpu.SemaphoreType.DMA,
        ],
    )
    msg, _alpha = f(q, k, v, src_flat, dst_flat, rp)
    return msg


def _prep_edges(src, dst):
    order = jnp.argsort(dst)
    dst_s = jnp.take(dst, order)
    src_s = jnp.take(src, order)
    bounds = jnp.searchsorted(dst_s, jnp.arange(NW + 1, dtype=jnp.int32) * NPW)
    bounds = bounds.astype(jnp.int32)
    cnt = bounds[1:] - bounds[:-1]
    pcnt = ((cnt + CE - 1) // CE) * CE
    starts = jnp.concatenate(
        [jnp.zeros((1,), jnp.int32), jnp.cumsum(pcnt).astype(jnp.int32)])

    p = jnp.arange(EPAD, dtype=jnp.int32)
    wb = jnp.clip(jnp.searchsorted(starts, p, side="right").astype(jnp.int32) - 1,
                  0, NW - 1)
    loc = p - starts[wb]
    valid = loc < cnt[wb]
    ep = jnp.clip(bounds[wb] + loc, 0, E - 1)
    src_flat = jnp.where(valid, jnp.take(src_s, ep), 0)
    dst_flat = jnp.where(valid, jnp.take(dst_s, ep), -1)

    rp = jnp.zeros((NW, 16), jnp.int32)
    rp = rp.at[:, 0].set(starts[:NW])
    rp = rp.at[:, 1].set(pcnt)
    return src_flat, dst_flat, rp


def kernel(x, edge_index,
           Wq0, bq0, Wk0, bk0, Wv0, bv0, Ws0, bs0,
           Wq1, bq1, Wk1, bk1, Wv1, bv1, Ws1, bs1,
           Wq2, bq2, Wk2, bk2, Wv2, bv2, Ws2, bs2):
    src = edge_index[0]
    dst = edge_index[1]
    src_flat, dst_flat, rp = _prep_edges(src, dst)

    def cat_w(Wq, Wk, Wv, Ws, bq, bk, bv, bs):
        w = jnp.concatenate([Wq, Wk, Wv, Ws], axis=1)
        b = jnp.concatenate([bq, bk, bv, bs], axis=0)
        b8 = jnp.zeros((8, CAT), jnp.float32).at[0].set(b)
        return w, b8

    w0, b0 = cat_w(Wq0, Wk0, Wv0, Ws0, bq0, bk0, bv0, bs0)
    w1, b1 = cat_w(Wq1, Wk1, Wv1, Ws1, bq1, bk1, bv1, bs1)
    w2, b2 = cat_w(Wq2, Wk2, Wv2, Ws2, bq2, bk2, bv2, bs2)

    xp = jnp.zeros((NPAD, D), jnp.float32).at[:N].set(x)

    q, k, v, s0 = _fused_mm(xp, w0, b0)
    msg0 = _edge_sc(q, k, v, src_flat, dst_flat, rp)

    q, k, v, s1 = _epi_fused_mm(msg0, s0, w1, b1)
    msg1 = _edge_sc(q, k, v, src_flat, dst_flat, rp)

    q, k, v, s2 = _epi_fused_mm(msg1, s1, w2, b2)
    msg2 = _edge_sc(q, k, v, src_flat, dst_flat, rp)

    return _final(msg2, s2)[:N]
